# baseline (speedup 1.0000x reference)
import jax
import jax.numpy as jnp
from jax.experimental import pallas as pl
from jax.experimental.pallas import tpu as pltpu

B, N, H, K = 8, 1024, 64, 9
NCHUNK = 16


def _body(idx_ref, k_ref, v_ref, m_ref, o_ref, buf, copy_sems, in_sems, out_sems):
    rows = N // (NCHUNK // B)

    for c in range(NCHUNK):
        b, s = divmod(c, NCHUNK // B)
        pltpu.make_async_copy(
            m_ref.at[b, pl.ds(s * rows, rows)],
            o_ref.at[b, pl.ds(s * rows, rows)],
            copy_sems.at[c],
        ).start()

    flat = lambda b, k: b * K + k

    def slot_info(b, k):
        idx = idx_ref[b, k]
        mult = 1
        first = True
        for j in range(K):
            if j == k:
                continue
            same = idx_ref[b, j] == idx
            mult = mult + same.astype(jnp.int32)
            if j < k:
                first = jnp.logical_and(first, jnp.logical_not(same))
        return idx, mult, first

    for b in range(B):
        for k in range(K):
            idx, mult, first = slot_info(b, k)
            bk = flat(b, k)

            @pl.when(first)
            def _(idx=idx, bk=bk, b=b):
                pltpu.make_async_copy(
                    m_ref.at[b, idx], buf.at[bk], in_sems.at[bk]
                ).start()

    for b in range(B):
        for k in range(K):
            idx, mult, first = slot_info(b, k)
            bk = flat(b, k)

            @pl.when(first)
            def _(idx=idx, mult=mult, bk=bk, b=b):
                pltpu.make_async_copy(
                    m_ref.at[b, idx], buf.at[bk], in_sems.at[bk]
                ).wait()
                krow = k_ref[b, idx, :] * mult.astype(jnp.float32)
                vrow = v_ref[b, idx, :]
                buf[bk] += krow[:, None] * vrow[None, :]

    for c in range(NCHUNK):
        b, s = divmod(c, NCHUNK // B)
        pltpu.make_async_copy(
            m_ref.at[b, pl.ds(s * rows, rows)],
            o_ref.at[b, pl.ds(s * rows, rows)],
            copy_sems.at[c],
        ).wait()

    for b in range(B):
        for k in range(K):
            idx, mult, first = slot_info(b, k)
            bk = flat(b, k)

            @pl.when(first)
            def _(idx=idx, bk=bk, b=b):
                pltpu.make_async_copy(
                    buf.at[bk], o_ref.at[b, idx], out_sems.at[bk]
                ).start()

    for b in range(B):
        for k in range(K):
            idx, mult, first = slot_info(b, k)
            bk = flat(b, k)

            @pl.when(first)
            def _(idx=idx, bk=bk, b=b):
                pltpu.make_async_copy(
                    buf.at[bk], o_ref.at[b, idx], out_sems.at[bk]
                ).wait()


@jax.jit
def kernel(M, M_k, M_v, indices_update):
    idx = indices_update.astype(jnp.int32)
    return pl.pallas_call(
        _body,
        in_specs=[
            pl.BlockSpec(memory_space=pltpu.SMEM),
            pl.BlockSpec(memory_space=pltpu.VMEM),
            pl.BlockSpec(memory_space=pltpu.VMEM),
            pl.BlockSpec(memory_space=pl.ANY),
        ],
        out_specs=pl.BlockSpec(memory_space=pl.ANY),
        out_shape=jax.ShapeDtypeStruct((B, N, H, H), jnp.float32),
        scratch_shapes=[
            pltpu.VMEM((B * K, H, H), jnp.float32),
            pltpu.SemaphoreType.DMA((NCHUNK,)),
            pltpu.SemaphoreType.DMA((B * K,)),
            pltpu.SemaphoreType.DMA((B * K,)),
        ],
    )(idx, M_k, M_v, M)


# speedup vs baseline: 65.4394x; 65.4394x over previous
import jax
import jax.numpy as jnp
from jax.experimental import pallas as pl
from jax.experimental.pallas import tpu as pltpu

B, N, H, K = 8, 1024, 64, 9
BN = 128


def _body(idx_ref, m_ref, k_ref, v_ref, o_ref):
    b = pl.program_id(0)
    j = pl.program_id(1)
    o_ref[...] = m_ref[...]
    lane_iota = jax.lax.broadcasted_iota(jnp.int32, (BN,), 0)
    for kk in range(K):
        idx = idx_ref[b, kk]
        inb = (idx >= j * BN) & (idx < (j + 1) * BN)

        @pl.when(inb)
        def _(idx=idx):
            lane = idx - j * BN
            onehot = (lane_iota == lane).astype(jnp.float32)
            kcol = jnp.sum(k_ref[0] * onehot[None, :], axis=-1)
            vcol = jnp.sum(v_ref[0] * onehot[None, :], axis=-1)
            outer = kcol[:, None] * vcol[None, :]
            o_ref[0] += outer[:, :, None] * onehot[None, None, :]


@jax.jit
def kernel(M, M_k, M_v, indices_update):
    idx = indices_update.astype(jnp.int32)
    Mt = jnp.transpose(M, (0, 2, 3, 1))
    Kt = jnp.transpose(M_k, (0, 2, 1))
    Vt = jnp.transpose(M_v, (0, 2, 1))
    out_t = pl.pallas_call(
        _body,
        grid=(B, N // BN),
        in_specs=[
            pl.BlockSpec(memory_space=pltpu.SMEM),
            pl.BlockSpec((1, H, H, BN), lambda b, j: (b, 0, 0, j)),
            pl.BlockSpec((1, H, BN), lambda b, j: (b, 0, j)),
            pl.BlockSpec((1, H, BN), lambda b, j: (b, 0, j)),
        ],
        out_specs=pl.BlockSpec((1, H, H, BN), lambda b, j: (b, 0, 0, j)),
        out_shape=jax.ShapeDtypeStruct((B, H, H, N), jnp.float32),
        compiler_params=pltpu.CompilerParams(
            dimension_semantics=("parallel", "parallel"),
        ),
    )(idx, Mt, Kt, Vt)
    return jnp.transpose(out_t, (0, 3, 1, 2))


# speedup vs baseline: 80.4030x; 1.2287x over previous
import jax
import jax.numpy as jnp
from jax.experimental import pallas as pl
from jax.experimental.pallas import tpu as pltpu

B, N, H, K = 8, 1024, 64, 9
BN = 256


def _body(idx_ref, m_ref, k_ref, v_ref, o_ref):
    b = pl.program_id(0)
    j = pl.program_id(1)
    o_ref[...] = m_ref[...]
    lane_iota = jax.lax.broadcasted_iota(jnp.int32, (BN,), 0)
    for kk in range(K):
        idx = idx_ref[b, kk]
        inb = (idx >= j * BN) & (idx < (j + 1) * BN)

        @pl.when(inb)
        def _(idx=idx):
            lane = idx - j * BN
            onehot = (lane_iota == lane).astype(jnp.float32)
            kcol = jnp.sum(k_ref[0] * onehot[None, :], axis=-1)
            vcol = jnp.sum(v_ref[0] * onehot[None, :], axis=-1)
            outer = kcol[:, None] * vcol[None, :]
            o_ref[0] += outer[:, :, None] * onehot[None, None, :]


@jax.jit
def kernel(M, M_k, M_v, indices_update):
    idx = indices_update.astype(jnp.int32)
    Mt = jnp.transpose(M, (0, 2, 3, 1))
    Kt = jnp.transpose(M_k, (0, 2, 1))
    Vt = jnp.transpose(M_v, (0, 2, 1))
    out_t = pl.pallas_call(
        _body,
        grid=(B, N // BN),
        in_specs=[
            pl.BlockSpec(memory_space=pltpu.SMEM),
            pl.BlockSpec((1, H, H, BN), lambda b, j: (b, 0, 0, j)),
            pl.BlockSpec((1, H, BN), lambda b, j: (b, 0, j)),
            pl.BlockSpec((1, H, BN), lambda b, j: (b, 0, j)),
        ],
        out_specs=pl.BlockSpec((1, H, H, BN), lambda b, j: (b, 0, 0, j)),
        out_shape=jax.ShapeDtypeStruct((B, H, H, N), jnp.float32),
        compiler_params=pltpu.CompilerParams(
            dimension_semantics=("parallel", "parallel"),
        ),
    )(idx, Mt, Kt, Vt)
    return jnp.transpose(out_t, (0, 3, 1, 2))
